# baseline (device time: 103832 ns/iter reference)
import jax
import jax.numpy as jnp
from jax import lax
from jax.experimental import pallas as pl
from jax.experimental.pallas import tpu as pltpu

N_DEV = 16


def kernel(x, Wq, Wo, K_ext, V_ext):
    B, Sq, D = x.shape
    Skv = K_ext.shape[1]
    Hl, Dh = K_ext.shape[2], K_ext.shape[3]
    Hd = Wq.shape[1]
    R = B * Sq
    CH = R // N_DEV

    x2 = x.reshape(R, D)
    K2 = K_ext.reshape(B, Skv, Hl * Dh)
    V2 = V_ext.reshape(B, Skv, Hl * Dh)

    def body(x_ref, wq_ref, wo_ref, k_ref, v_ref, out_ref,
             attn_ref, rs_send, rs_recv, ag_send, ag_recv,
             rs_s_sems, rs_r_sems, ag_s_sems, ag_r_sems):
        my = lax.axis_index("i")
        left = lax.rem(my + N_DEV - 1, N_DEV)
        right = lax.rem(my + 1, N_DEV)

        barrier = pltpu.get_barrier_semaphore()
        for nbr in (left, right):
            pl.semaphore_signal(barrier, inc=1, device_id=(nbr,),
                                device_id_type=pl.DeviceIdType.MESH)
        pl.semaphore_wait(barrier, 2)

        xb = x_ref[:].astype(jnp.bfloat16)
        wq = wq_ref[:].astype(jnp.bfloat16)
        q_all = jnp.dot(xb, wq, preferred_element_type=jnp.float32)
        for b in range(B):
            for h in range(Hl):
                q = q_all[b * Sq:(b + 1) * Sq,
                          h * Dh:(h + 1) * Dh].astype(jnp.bfloat16)
                k = k_ref[b, :, h * Dh:(h + 1) * Dh].astype(jnp.bfloat16)
                v = v_ref[b, :, h * Dh:(h + 1) * Dh].astype(jnp.bfloat16)
                s = lax.dot_general(q, k, (((1,), (1,)), ((), ())),
                                    preferred_element_type=jnp.float32)
                s = s * 0.125
                m = jnp.max(s, axis=-1, keepdims=True)
                p = jnp.exp(s - m)
                l = jnp.sum(p, axis=-1, keepdims=True)
                o = jnp.dot(p.astype(jnp.bfloat16), v,
                            preferred_element_type=jnp.float32)
                attn_ref[b * Sq:(b + 1) * Sq,
                         h * Dh:(h + 1) * Dh] = (o / l).astype(jnp.bfloat16)
        wo = wo_ref[:].astype(jnp.bfloat16)
        out_ref[:] = jnp.dot(attn_ref[:], wo,
                             preferred_element_type=jnp.float32)

        for si in range(N_DEV - 1):
            c = lax.rem(my + (N_DEV - si), N_DEV)
            val = out_ref[pl.ds(c * CH, CH), :]
            if si > 0:
                val = val + rs_recv[si - 1, :, :]
            rs_send[si, :, :] = val
            rdma = pltpu.make_async_remote_copy(
                src_ref=rs_send.at[si],
                dst_ref=rs_recv.at[si],
                send_sem=rs_s_sems.at[si],
                recv_sem=rs_r_sems.at[si],
                device_id=(right,),
                device_id_type=pl.DeviceIdType.MESH,
            )
            rdma.start()
            rdma.wait()

        cr = lax.rem(my + 1, N_DEV)
        red = out_ref[pl.ds(cr * CH, CH), :] + rs_recv[N_DEV - 2, :, :]
        out_ref[pl.ds(cr * CH, CH), :] = red
        ag_send[:] = red

        for t in range(N_DEV - 1):
            src = ag_send if t == 0 else ag_recv.at[t - 1]
            rdma = pltpu.make_async_remote_copy(
                src_ref=src,
                dst_ref=ag_recv.at[t],
                send_sem=ag_s_sems.at[t],
                recv_sem=ag_r_sems.at[t],
                device_id=(right,),
                device_id_type=pl.DeviceIdType.MESH,
            )
            rdma.start()
            rdma.wait()
            g = lax.rem(my + (N_DEV - t), N_DEV)
            out_ref[pl.ds(g * CH, CH), :] = ag_recv[t, :, :]

    out = pl.pallas_call(
        body,
        out_shape=jax.ShapeDtypeStruct((R, D), jnp.float32),
        in_specs=[pl.BlockSpec(memory_space=pltpu.VMEM)] * 5,
        out_specs=pl.BlockSpec(memory_space=pltpu.VMEM),
        scratch_shapes=[
            pltpu.VMEM((R, Hd), jnp.bfloat16),
            pltpu.VMEM((N_DEV - 1, CH, D), jnp.float32),
            pltpu.VMEM((N_DEV - 1, CH, D), jnp.float32),
            pltpu.VMEM((CH, D), jnp.float32),
            pltpu.VMEM((N_DEV - 1, CH, D), jnp.float32),
            pltpu.SemaphoreType.DMA((N_DEV - 1,)),
            pltpu.SemaphoreType.DMA((N_DEV - 1,)),
            pltpu.SemaphoreType.DMA((N_DEV - 1,)),
            pltpu.SemaphoreType.DMA((N_DEV - 1,)),
        ],
        compiler_params=pltpu.CompilerParams(collective_id=0),
    )(x2, Wq, Wo, K2, V2)
    return out.reshape(B, Sq, D)


# device time: 36524 ns/iter; 2.8428x vs baseline; 2.8428x over previous
import jax
import jax.numpy as jnp
from jax import lax
from jax.experimental import pallas as pl
from jax.experimental.pallas import tpu as pltpu

N_DEV = 16


def kernel(x, Wq, Wo, K_ext, V_ext):
    B, Sq, D = x.shape
    Skv = K_ext.shape[1]
    Hl, Dh = K_ext.shape[2], K_ext.shape[3]
    Hd = Wq.shape[1]
    R = B * Sq
    CH = R // N_DEV

    x2 = x.reshape(R, D)
    K2 = K_ext.reshape(B, Skv, Hl * Dh)
    V2 = V_ext.reshape(B, Skv, Hl * Dh)

    def body(x_ref, wq_ref, wo_ref, k_ref, v_ref, out_ref,
             attn_ref, part_ref, rs_recv, ag_send, ag_recv,
             rs_s_sems, rs_r_sems, ag_s_sems, ag_r_sems):
        my = lax.axis_index("i")

        barrier = pltpu.get_barrier_semaphore()
        for e in range(N_DEV):
            @pl.when(e != my)
            def _():
                pl.semaphore_signal(barrier, inc=1, device_id=(e,),
                                    device_id_type=pl.DeviceIdType.MESH)
        pl.semaphore_wait(barrier, N_DEV - 1)

        xb = x_ref[:].astype(jnp.bfloat16)
        wq = wq_ref[:].astype(jnp.bfloat16)
        q_all = jnp.dot(xb, wq, preferred_element_type=jnp.float32)
        for b in range(B):
            for h in range(Hl):
                q = q_all[b * Sq:(b + 1) * Sq,
                          h * Dh:(h + 1) * Dh].astype(jnp.bfloat16)
                k = k_ref[b, :, h * Dh:(h + 1) * Dh].astype(jnp.bfloat16)
                v = v_ref[b, :, h * Dh:(h + 1) * Dh].astype(jnp.bfloat16)
                s = lax.dot_general(q, k, (((1,), (1,)), ((), ())),
                                    preferred_element_type=jnp.float32)
                s = s * 0.125
                m = jnp.max(s, axis=-1, keepdims=True)
                p = jnp.exp(s - m)
                l = jnp.sum(p, axis=-1, keepdims=True)
                o = jnp.dot(p.astype(jnp.bfloat16), v,
                            preferred_element_type=jnp.float32)
                attn_ref[b * Sq:(b + 1) * Sq,
                         h * Dh:(h + 1) * Dh] = (o / l).astype(jnp.bfloat16)
        wo = wo_ref[:].astype(jnp.bfloat16)
        partial = jnp.dot(attn_ref[:], wo,
                          preferred_element_type=jnp.float32)
        out_ref[:] = partial
        for c in range(N_DEV):
            part_ref[c, :, :] = partial[c * CH:(c + 1) * CH, :].astype(
                jnp.bfloat16)

        for e in range(N_DEV):
            @pl.when(e != my)
            def _():
                rdma = pltpu.make_async_remote_copy(
                    src_ref=part_ref.at[e],
                    dst_ref=rs_recv.at[my],
                    send_sem=rs_s_sems.at[e],
                    recv_sem=rs_r_sems.at[my],
                    device_id=(e,),
                    device_id_type=pl.DeviceIdType.MESH,
                )
                rdma.start()

        for j in range(N_DEV):
            @pl.when(j != my)
            def _():
                recv = pltpu.make_async_remote_copy(
                    src_ref=part_ref.at[j],
                    dst_ref=rs_recv.at[j],
                    send_sem=rs_s_sems.at[j],
                    recv_sem=rs_r_sems.at[j],
                    device_id=(j,),
                    device_id_type=pl.DeviceIdType.MESH,
                )
                recv.wait_recv()
        red = out_ref[pl.ds(my * CH, CH), :]
        for j in range(N_DEV):
            c = rs_recv[j, :, :].astype(jnp.float32)
            red = red + jnp.where(my == j, jnp.zeros_like(c), c)

        ag_send[:] = red.astype(jnp.bfloat16)
        for e in range(N_DEV):
            @pl.when(e != my)
            def _():
                rdma = pltpu.make_async_remote_copy(
                    src_ref=ag_send,
                    dst_ref=ag_recv.at[my],
                    send_sem=ag_s_sems.at[e],
                    recv_sem=ag_r_sems.at[my],
                    device_id=(e,),
                    device_id_type=pl.DeviceIdType.MESH,
                )
                rdma.start()

        for j in range(N_DEV):
            @pl.when(j != my)
            def _():
                recv = pltpu.make_async_remote_copy(
                    src_ref=ag_send,
                    dst_ref=ag_recv.at[j],
                    send_sem=ag_s_sems.at[j],
                    recv_sem=ag_r_sems.at[j],
                    device_id=(j,),
                    device_id_type=pl.DeviceIdType.MESH,
                )
                recv.wait_recv()

        for j in range(N_DEV):
            val = jnp.where(my == j, red,
                            ag_recv[j, :, :].astype(jnp.float32))
            out_ref[j * CH:(j + 1) * CH, :] = val

        for e in range(N_DEV):
            @pl.when(e != my)
            def _():
                s1 = pltpu.make_async_remote_copy(
                    src_ref=part_ref.at[e], dst_ref=rs_recv.at[e],
                    send_sem=rs_s_sems.at[e], recv_sem=rs_r_sems.at[e],
                    device_id=(e,), device_id_type=pl.DeviceIdType.MESH,
                )
                s1.wait_send()
                s2 = pltpu.make_async_remote_copy(
                    src_ref=ag_send, dst_ref=ag_recv.at[e],
                    send_sem=ag_s_sems.at[e], recv_sem=ag_r_sems.at[e],
                    device_id=(e,), device_id_type=pl.DeviceIdType.MESH,
                )
                s2.wait_send()

    out = pl.pallas_call(
        body,
        out_shape=jax.ShapeDtypeStruct((R, D), jnp.float32),
        in_specs=[pl.BlockSpec(memory_space=pltpu.VMEM)] * 5,
        out_specs=pl.BlockSpec(memory_space=pltpu.VMEM),
        scratch_shapes=[
            pltpu.VMEM((R, Hd), jnp.bfloat16),
            pltpu.VMEM((N_DEV, CH, D), jnp.bfloat16),
            pltpu.VMEM((N_DEV, CH, D), jnp.bfloat16),
            pltpu.VMEM((CH, D), jnp.bfloat16),
            pltpu.VMEM((N_DEV, CH, D), jnp.bfloat16),
            pltpu.SemaphoreType.DMA((N_DEV,)),
            pltpu.SemaphoreType.DMA((N_DEV,)),
            pltpu.SemaphoreType.DMA((N_DEV,)),
            pltpu.SemaphoreType.DMA((N_DEV,)),
        ],
        compiler_params=pltpu.CompilerParams(collective_id=0),
    )(x2, Wq, Wo, K2, V2)
    return out.reshape(B, Sq, D)
